# baseline (speedup 1.0000x reference)
import functools

import jax
import jax.numpy as jnp
from jax import lax
from jax.experimental import pallas as pl
from jax.experimental.pallas import tpu as pltpu
from jax.experimental.pallas import tpu_sc as plsc

S, D, E, H, O = 4096, 1024, 64, 512, 1024
CAP = (2 * S) // E
TB = 256
NB = S // TB

NC, NS, L = 2, 16, 16
NW = NC * NS
SLOTS = E * CAP
SPW = SLOTS // NW
TPW = S // NW
GCH = 16
CCH = 16

_MESH = plsc.VectorSubcoreMesh(
    core_axis_name="c", subcore_axis_name="s", num_cores=NC, num_subcores=NS)


def _route_body(x_ref, wg_ref,
                idx1_ref, idx2_ref, loc1_ref, loc2p_ref, g1_ref, g2_ref,
                c1_ref, laux_ref,
                c1s, c2s, gs):
    i = pl.program_id(0)

    @pl.when(i == 0)
    def _init():
        c1s[...] = jnp.zeros_like(c1s)
        c2s[...] = jnp.zeros_like(c2s)
        gs[...] = jnp.zeros_like(gs)

    x = x_ref[...]
    wg = wg_ref[...]
    logits = jnp.dot(x, wg, preferred_element_type=jnp.float32)
    mx1 = jnp.max(logits, axis=1, keepdims=True)
    p = jnp.exp(logits - mx1)
    gates = p / jnp.sum(p, axis=1, keepdims=True)

    iota_e = lax.broadcasted_iota(jnp.int32, (TB, E), 1)
    idx1 = jnp.min(jnp.where(logits == mx1, iota_e, E), axis=1)
    g1 = jnp.max(gates, axis=1)
    m1 = iota_e == idx1[:, None]
    l2 = jnp.where(m1, -1e9, logits)
    mx2 = jnp.max(l2, axis=1, keepdims=True)
    idx2 = jnp.min(jnp.where(l2 == mx2, iota_e, E), axis=1)
    g2 = jnp.max(jnp.where(m1, 0.0, gates), axis=1)
    m2 = iota_e == idx2[:, None]

    m1f = m1.astype(jnp.float32)
    m2f = m2.astype(jnp.float32)
    r_io = lax.broadcasted_iota(jnp.int32, (TB, TB), 0)
    c_io = lax.broadcasted_iota(jnp.int32, (TB, TB), 1)
    tril = (r_io >= c_io).astype(jnp.float32)
    cum1 = jnp.dot(tril, m1f, preferred_element_type=jnp.float32)
    cum2 = jnp.dot(tril, m2f, preferred_element_type=jnp.float32)
    prior1 = c1s[0]
    prior2 = c2s[0]
    loc1 = jnp.sum((cum1 + prior1[None, :]) * m1f, axis=1) - 1.0
    loc2p = jnp.sum((cum2 + prior2[None, :]) * m2f, axis=1) - 1.0

    idx1_ref[0] = idx1.reshape(TB // 128, 128)
    idx2_ref[0] = idx2.reshape(TB // 128, 128)
    loc1_ref[0] = loc1.astype(jnp.int32).reshape(TB // 128, 128)
    loc2p_ref[0] = loc2p.astype(jnp.int32).reshape(TB // 128, 128)
    g1_ref[0] = g1.reshape(TB // 128, 128)
    g2_ref[0] = g2.reshape(TB // 128, 128)

    c1s[0] = prior1 + jnp.sum(m1f, axis=0)
    c2s[0] = prior2 + jnp.sum(m2f, axis=0)
    gs[0] = gs[0] + jnp.sum(gates, axis=0)

    @pl.when(i == NB - 1)
    def _fin():
        me = gs[0] / S
        ce = c1s[0] / S
        laux_ref[0, 0] = jnp.sum(me * ce) * E
        c1_ref[...] = c1s[...].astype(jnp.int32)


def _route(x, wg):
    return pl.pallas_call(
        _route_body,
        grid=(NB,),
        in_specs=[
            pl.BlockSpec((TB, D), lambda i: (i, 0)),
            pl.BlockSpec((D, E), lambda i: (0, 0)),
        ],
        out_specs=[
            pl.BlockSpec((1, TB // 128, 128), lambda i: (i, 0, 0)),
            pl.BlockSpec((1, TB // 128, 128), lambda i: (i, 0, 0)),
            pl.BlockSpec((1, TB // 128, 128), lambda i: (i, 0, 0)),
            pl.BlockSpec((1, TB // 128, 128), lambda i: (i, 0, 0)),
            pl.BlockSpec((1, TB // 128, 128), lambda i: (i, 0, 0)),
            pl.BlockSpec((1, TB // 128, 128), lambda i: (i, 0, 0)),
            pl.BlockSpec((1, E), lambda i: (0, 0)),
            pl.BlockSpec(memory_space=pltpu.SMEM),
        ],
        out_shape=[
            jax.ShapeDtypeStruct((NB, TB // 128, 128), jnp.int32),
            jax.ShapeDtypeStruct((NB, TB // 128, 128), jnp.int32),
            jax.ShapeDtypeStruct((NB, TB // 128, 128), jnp.int32),
            jax.ShapeDtypeStruct((NB, TB // 128, 128), jnp.int32),
            jax.ShapeDtypeStruct((NB, TB // 128, 128), jnp.float32),
            jax.ShapeDtypeStruct((NB, TB // 128, 128), jnp.float32),
            jax.ShapeDtypeStruct((1, E), jnp.int32),
            jax.ShapeDtypeStruct((1, 1), jnp.float32),
        ],
        scratch_shapes=[
            pltpu.VMEM((1, E), jnp.float32),
            pltpu.VMEM((1, E), jnp.float32),
            pltpu.VMEM((1, E), jnp.float32),
        ],
    )(x, wg)


def _ffn_issue(tok_ref, x_any, xb, sem, eidx):
    base = eidx * CAP
    for i in range(CAP):
        t = tok_ref[base + i]
        pltpu.make_async_copy(
            x_any.at[pl.ds(t, 1)], xb.at[pl.ds(i, 1)], sem).start()


def _ffn_body(tok_ref, x_any, w1_ref, b1_ref, w2_ref, b2_ref, eo_ref,
              xb0, xb1, sem0, sem1):
    e = pl.program_id(0)
    even = lax.rem(e, 2) == 0

    @pl.when(e == 0)
    def _prime():
        _ffn_issue(tok_ref, x_any, xb0, sem0, e)

    def _step(xb, sem, xbn, semn):
        pltpu.make_async_copy(x_any.at[pl.ds(0, CAP)], xb, sem).wait()
        en = jnp.minimum(e + 1, E - 1)
        _ffn_issue(tok_ref, x_any, xbn, semn, en)
        d = xb[...]
        h = jnp.dot(d, w1_ref[0], preferred_element_type=jnp.float32)
        h = jnp.maximum(h + b1_ref[0], 0.0)
        eo = jnp.dot(h, w2_ref[0], preferred_element_type=jnp.float32)
        eo_ref[0] = eo + b2_ref[0]

    @pl.when(even)
    def _c0():
        _step(xb0, sem0, xb1, sem1)

    @pl.when(jnp.logical_not(even))
    def _c1():
        _step(xb1, sem1, xb0, sem0)

    @pl.when(e == E - 1)
    def _drain():
        pltpu.make_async_copy(x_any.at[pl.ds(0, CAP)], xb0, sem0).wait()


def _ffn(slot_tok, x, W1, b1, W2, b2):
    return pl.pallas_call(
        _ffn_body,
        grid=(E,),
        in_specs=[
            pl.BlockSpec(memory_space=pltpu.SMEM),
            pl.BlockSpec(memory_space=pl.ANY),
            pl.BlockSpec((1, D, H), lambda e: (e, 0, 0)),
            pl.BlockSpec((1, 1, H), lambda e: (e, 0, 0)),
            pl.BlockSpec((1, H, O), lambda e: (e, 0, 0)),
            pl.BlockSpec((1, 1, O), lambda e: (e, 0, 0)),
        ],
        out_specs=pl.BlockSpec((1, CAP, O), lambda e: (e, 0, 0)),
        out_shape=jax.ShapeDtypeStruct((E, CAP, O), jnp.float32),
        scratch_shapes=[
            pltpu.VMEM((CAP, D), jnp.float32),
            pltpu.VMEM((CAP, D), jnp.float32),
            pltpu.SemaphoreType.DMA,
            pltpu.SemaphoreType.DMA,
        ],
    )(slot_tok, x, W1, b1.reshape(E, 1, H), W2, b2.reshape(E, 1, O))


def _dispatch_body(idx1_h, idx2_h, loc1_h, loc2p_h, g1_h, g2_h, c1_h,
                   tok_h, pos1_h, pos2_h, s1_h, s2_h,
                   idx1_v, idx2_v, loc1_v, loc2p_v, g1_v, g2_v, c1_v,
                   table, p1b, p2b, s1b, s2b):
    wid = lax.axis_index("s") * NC + lax.axis_index("c")
    slot_lo = wid * SPW
    tok_lo = wid * TPW

    pltpu.sync_copy(idx1_h, idx1_v)
    pltpu.sync_copy(idx2_h, idx2_v)
    pltpu.sync_copy(loc1_h, loc1_v)
    pltpu.sync_copy(loc2p_h, loc2p_v)
    pltpu.sync_copy(g1_h.at[pl.ds(tok_lo, TPW)], g1_v)
    pltpu.sync_copy(g2_h.at[pl.ds(tok_lo, TPW)], g2_v)
    pltpu.sync_copy(c1_h, c1_v)

    def _init(j, _):
        table[pl.ds(j * L, L)] = jnp.zeros((L,), jnp.int32)
        return 0
    lax.fori_loop(0, SPW // L, _init, 0)

    lanes = lax.iota(jnp.int32, L)

    def _scan(j, _):
        sl = pl.ds(j * L, L)
        tvec = lanes + j * L
        i1 = idx1_v[sl]
        l1 = loc1_v[sl]
        p1 = i1 * CAP + l1
        m1 = (l1 < CAP) & (p1 >= slot_lo) & (p1 < slot_lo + SPW)
        plsc.store_scatter(table, [jnp.where(m1, p1 - slot_lo, 0)], tvec,
                           mask=m1)
        i2 = idx2_v[sl]
        l2 = loc2p_v[sl] + plsc.load_gather(c1_v, [i2])
        p2 = i2 * CAP + l2
        m2 = (l2 < CAP) & (p2 >= slot_lo) & (p2 < slot_lo + SPW)
        plsc.store_scatter(table, [jnp.where(m2, p2 - slot_lo, 0)], tvec,
                           mask=m2)
        return 0
    lax.fori_loop(0, S // L, _scan, 0)

    def _scales(j, _):
        slg = pl.ds(tok_lo + j * L, L)
        sll = pl.ds(j * L, L)
        i1 = idx1_v[slg]
        l1 = loc1_v[slg]
        i2 = idx2_v[slg]
        l2 = loc2p_v[slg] + plsc.load_gather(c1_v, [i2])
        g1 = g1_v[sll]
        g2 = g2_v[sll]
        k1 = l1 < CAP
        k2 = l2 < CAP
        g1k = jnp.where(k1, g1, 0.0)
        g2k = jnp.where(k2, g2, 0.0)
        denom = jnp.maximum(g1k + g2k, 1e-9)
        p1b[sll] = i1 * CAP + jnp.where(k1, l1, 0)
        p2b[sll] = i2 * CAP + jnp.where(k2, l2, 0)
        s1b[sll] = g1k / denom
        s2b[sll] = g2k / denom
        return 0
    lax.fori_loop(0, TPW // L, _scales, 0)
    pltpu.sync_copy(p1b, pos1_h.at[pl.ds(tok_lo, TPW)])
    pltpu.sync_copy(p2b, pos2_h.at[pl.ds(tok_lo, TPW)])
    pltpu.sync_copy(s1b, s1_h.at[pl.ds(tok_lo, TPW)])
    pltpu.sync_copy(s2b, s2_h.at[pl.ds(tok_lo, TPW)])

    pltpu.sync_copy(table, tok_h.at[pl.ds(slot_lo, SPW)])


def _dispatch(idx1, idx2, loc1, loc2p, g1, g2, c1):
    f = functools.partial(
        pl.kernel,
        out_type=(
            jax.ShapeDtypeStruct((SLOTS,), jnp.int32),
            jax.ShapeDtypeStruct((S,), jnp.int32),
            jax.ShapeDtypeStruct((S,), jnp.int32),
            jax.ShapeDtypeStruct((S,), jnp.float32),
            jax.ShapeDtypeStruct((S,), jnp.float32),
        ),
        mesh=_MESH,
        scratch_types=[
            pltpu.VMEM((S,), jnp.int32),
            pltpu.VMEM((S,), jnp.int32),
            pltpu.VMEM((S,), jnp.int32),
            pltpu.VMEM((S,), jnp.int32),
            pltpu.VMEM((TPW,), jnp.float32),
            pltpu.VMEM((TPW,), jnp.float32),
            pltpu.VMEM((E,), jnp.int32),
            pltpu.VMEM((SPW,), jnp.int32),
            pltpu.VMEM((TPW,), jnp.int32),
            pltpu.VMEM((TPW,), jnp.int32),
            pltpu.VMEM((TPW,), jnp.float32),
            pltpu.VMEM((TPW,), jnp.float32),
        ],
        compiler_params=pltpu.CompilerParams(needs_layout_passes=False),
    )(_dispatch_body)
    return f(idx1, idx2, loc1, loc2p, g1, g2, c1)


def _combine_body(eo_h, pos1_h, pos2_h, s1_h, s2_h, out_h,
                  p1v, p2v, s1v, s2v, r1, r2, ob, gsem, osem):
    wid = lax.axis_index("s") * NC + lax.axis_index("c")
    tok_lo = wid * TPW

    pltpu.sync_copy(pos1_h.at[pl.ds(tok_lo, TPW)], p1v)
    pltpu.sync_copy(pos2_h.at[pl.ds(tok_lo, TPW)], p2v)
    pltpu.sync_copy(s1_h.at[pl.ds(tok_lo, TPW)], s1v)
    pltpu.sync_copy(s2_h.at[pl.ds(tok_lo, TPW)], s2v)

    nc = TPW // CCH
    g1cp = [None] * 2
    g2cp = [None] * 2
    ocp = [None] * 2
    g1cp[0] = pltpu.async_copy(eo_h.at[p1v[pl.ds(0, CCH)]], r1.at[0], gsem)
    g2cp[0] = pltpu.async_copy(eo_h.at[p2v[pl.ds(0, CCH)]], r2.at[0], gsem)
    for ch in range(nc):
        b = ch % 2
        if ch + 1 < nc:
            nb = 1 - b
            g1cp[nb] = pltpu.async_copy(
                eo_h.at[p1v[pl.ds((ch + 1) * CCH, CCH)]], r1.at[nb], gsem)
            g2cp[nb] = pltpu.async_copy(
                eo_h.at[p2v[pl.ds((ch + 1) * CCH, CCH)]], r2.at[nb], gsem)
        g1cp[b].wait()
        g2cp[b].wait()
        if ocp[b] is not None:
            ocp[b].wait()

        def _tok(t, _):
            ti = jnp.full((L,), ch * CCH + t, jnp.int32)
            sa = plsc.load_gather(s1v, [ti])
            sb = plsc.load_gather(s2v, [ti])

            def _col(v, __):
                sl = pl.ds(v * L, L)
                ob[b, t, sl] = sa * r1[b, t, sl] + sb * r2[b, t, sl]
                return 0
            lax.fori_loop(0, O // L, _col, 0)
            return 0
        lax.fori_loop(0, CCH, _tok, 0)
        ocp[b] = pltpu.async_copy(
            ob.at[b], out_h.at[pl.ds(tok_lo + ch * CCH, CCH)], osem)
    for b in range(2):
        if ocp[b] is not None:
            ocp[b].wait()


def _combine(eo, pos1, pos2, s1, s2):
    f = functools.partial(
        pl.kernel,
        out_type=jax.ShapeDtypeStruct((S, O), jnp.float32),
        mesh=_MESH,
        scratch_types=[
            pltpu.VMEM((TPW,), jnp.int32),
            pltpu.VMEM((TPW,), jnp.int32),
            pltpu.VMEM((TPW,), jnp.float32),
            pltpu.VMEM((TPW,), jnp.float32),
            pltpu.VMEM((2, CCH, O), jnp.float32),
            pltpu.VMEM((2, CCH, O), jnp.float32),
            pltpu.VMEM((2, CCH, O), jnp.float32),
            pltpu.SemaphoreType.DMA,
            pltpu.SemaphoreType.DMA,
        ],
        compiler_params=pltpu.CompilerParams(needs_layout_passes=False),
    )(_combine_body)
    return f(eo, pos1, pos2, s1, s2)


def kernel(x, wg, W1, b1, W2, b2):
    idx1, idx2, loc1, loc2p, g1, g2, c1, laux = _route(x, wg)
    slot_tok, pos1, pos2, s1, s2 = _dispatch(
        idx1.reshape(S), idx2.reshape(S), loc1.reshape(S), loc2p.reshape(S),
        g1.reshape(S), g2.reshape(S), c1.reshape(E))
    eo = _ffn(slot_tok, x, W1, b1, W2, b2)
    out = _combine(eo.reshape(SLOTS, O), pos1, pos2, s1, s2)
    return out, laux[0, 0]


# speedup vs baseline: 1.1126x; 1.1126x over previous
import functools

import jax
import jax.numpy as jnp
from jax import lax
from jax.experimental import pallas as pl
from jax.experimental.pallas import tpu as pltpu
from jax.experimental.pallas import tpu_sc as plsc

S, D, E, H, O = 4096, 1024, 64, 512, 1024
CAP = (2 * S) // E
TB = 256
NB = S // TB

NC, NS, L = 2, 16, 16
NW = NC * NS
SLOTS = E * CAP
SPW = SLOTS // NW
TPW = S // NW
GCH = 16
CCH = 16

_MESH = plsc.VectorSubcoreMesh(
    core_axis_name="c", subcore_axis_name="s", num_cores=NC, num_subcores=NS)


def _route_body(x_ref, wg_ref,
                idx1_ref, idx2_ref, loc1_ref, loc2p_ref, g1_ref, g2_ref,
                c1_ref, laux_ref,
                c1s, c2s, gs):
    i = pl.program_id(0)

    @pl.when(i == 0)
    def _init():
        c1s[...] = jnp.zeros_like(c1s)
        c2s[...] = jnp.zeros_like(c2s)
        gs[...] = jnp.zeros_like(gs)

    x = x_ref[...]
    wg = wg_ref[...]
    logits = jnp.dot(x, wg, preferred_element_type=jnp.float32)
    mx1 = jnp.max(logits, axis=1, keepdims=True)
    p = jnp.exp(logits - mx1)
    gates = p / jnp.sum(p, axis=1, keepdims=True)

    iota_e = lax.broadcasted_iota(jnp.int32, (TB, E), 1)
    idx1 = jnp.min(jnp.where(logits == mx1, iota_e, E), axis=1)
    g1 = jnp.max(gates, axis=1)
    m1 = iota_e == idx1[:, None]
    l2 = jnp.where(m1, -1e9, logits)
    mx2 = jnp.max(l2, axis=1, keepdims=True)
    idx2 = jnp.min(jnp.where(l2 == mx2, iota_e, E), axis=1)
    g2 = jnp.max(jnp.where(m1, 0.0, gates), axis=1)
    m2 = iota_e == idx2[:, None]

    m1f = m1.astype(jnp.float32)
    m2f = m2.astype(jnp.float32)
    r_io = lax.broadcasted_iota(jnp.int32, (TB, TB), 0)
    c_io = lax.broadcasted_iota(jnp.int32, (TB, TB), 1)
    tril = (r_io >= c_io).astype(jnp.float32)
    cum1 = jnp.dot(tril, m1f, preferred_element_type=jnp.float32)
    cum2 = jnp.dot(tril, m2f, preferred_element_type=jnp.float32)
    prior1 = c1s[0]
    prior2 = c2s[0]
    loc1 = jnp.sum((cum1 + prior1[None, :]) * m1f, axis=1) - 1.0
    loc2p = jnp.sum((cum2 + prior2[None, :]) * m2f, axis=1) - 1.0

    idx1_ref[0] = idx1.reshape(TB // 128, 128)
    idx2_ref[0] = idx2.reshape(TB // 128, 128)
    loc1_ref[0] = loc1.astype(jnp.int32).reshape(TB // 128, 128)
    loc2p_ref[0] = loc2p.astype(jnp.int32).reshape(TB // 128, 128)
    g1_ref[0] = g1.reshape(TB // 128, 128)
    g2_ref[0] = g2.reshape(TB // 128, 128)

    c1s[0] = prior1 + jnp.sum(m1f, axis=0)
    c2s[0] = prior2 + jnp.sum(m2f, axis=0)
    gs[0] = gs[0] + jnp.sum(gates, axis=0)

    @pl.when(i == NB - 1)
    def _fin():
        me = gs[0] / S
        ce = c1s[0] / S
        laux_ref[0, 0] = jnp.sum(me * ce) * E
        c1_ref[...] = c1s[...].astype(jnp.int32)


def _route(x, wg):
    return pl.pallas_call(
        _route_body,
        grid=(NB,),
        in_specs=[
            pl.BlockSpec((TB, D), lambda i: (i, 0)),
            pl.BlockSpec((D, E), lambda i: (0, 0)),
        ],
        out_specs=[
            pl.BlockSpec((1, TB // 128, 128), lambda i: (i, 0, 0)),
            pl.BlockSpec((1, TB // 128, 128), lambda i: (i, 0, 0)),
            pl.BlockSpec((1, TB // 128, 128), lambda i: (i, 0, 0)),
            pl.BlockSpec((1, TB // 128, 128), lambda i: (i, 0, 0)),
            pl.BlockSpec((1, TB // 128, 128), lambda i: (i, 0, 0)),
            pl.BlockSpec((1, TB // 128, 128), lambda i: (i, 0, 0)),
            pl.BlockSpec((1, E), lambda i: (0, 0)),
            pl.BlockSpec(memory_space=pltpu.SMEM),
        ],
        out_shape=[
            jax.ShapeDtypeStruct((NB, TB // 128, 128), jnp.int32),
            jax.ShapeDtypeStruct((NB, TB // 128, 128), jnp.int32),
            jax.ShapeDtypeStruct((NB, TB // 128, 128), jnp.int32),
            jax.ShapeDtypeStruct((NB, TB // 128, 128), jnp.int32),
            jax.ShapeDtypeStruct((NB, TB // 128, 128), jnp.float32),
            jax.ShapeDtypeStruct((NB, TB // 128, 128), jnp.float32),
            jax.ShapeDtypeStruct((1, E), jnp.int32),
            jax.ShapeDtypeStruct((1, 1), jnp.float32),
        ],
        scratch_shapes=[
            pltpu.VMEM((1, E), jnp.float32),
            pltpu.VMEM((1, E), jnp.float32),
            pltpu.VMEM((1, E), jnp.float32),
        ],
    )(x, wg)


def _ffn_issue(tok_ref, x_any, xb, sem, eidx):
    base = eidx * CAP
    for i in range(CAP):
        t = tok_ref[base + i]
        pltpu.make_async_copy(
            x_any.at[pl.ds(t, 1)], xb.at[pl.ds(i, 1)], sem).start()


def _ffn_body(tok_ref, x_any, w1_ref, b1_ref, w2_ref, b2_ref, eo_ref,
              xb0, xb1, xb2, sem0, sem1, sem2):
    e = pl.program_id(0)
    xbs = (xb0, xb1, xb2)
    sems = (sem0, sem1, sem2)

    @pl.when(e == 0)
    def _prime():
        _ffn_issue(tok_ref, x_any, xb0, sem0, e)
        _ffn_issue(tok_ref, x_any, xb1, sem1, e + 1)

    def _step(b):
        en = jnp.minimum(e + 2, E - 1)
        _ffn_issue(tok_ref, x_any, xbs[(b + 2) % 3], sems[(b + 2) % 3], en)
        xb = xbs[b]
        pltpu.make_async_copy(x_any.at[pl.ds(0, CAP)], xb, sems[b]).wait()
        d = xb[...]
        h = jnp.dot(d, w1_ref[0], preferred_element_type=jnp.float32)
        h = jnp.maximum(h + b1_ref[0], 0.0)
        eo = jnp.dot(h, w2_ref[0], preferred_element_type=jnp.float32)
        eo_ref[0] = eo + b2_ref[0]

    r3 = lax.rem(e, 3)
    for b in range(3):
        @pl.when(r3 == b)
        def _c(b=b):
            _step(b)

    @pl.when(e == E - 1)
    def _drain():
        pltpu.make_async_copy(x_any.at[pl.ds(0, CAP)], xbs[E % 3],
                              sems[E % 3]).wait()
        pltpu.make_async_copy(x_any.at[pl.ds(0, CAP)], xbs[(E + 1) % 3],
                              sems[(E + 1) % 3]).wait()


def _ffn(slot_tok, x, W1, b1, W2, b2):
    return pl.pallas_call(
        _ffn_body,
        grid=(E,),
        in_specs=[
            pl.BlockSpec(memory_space=pltpu.SMEM),
            pl.BlockSpec(memory_space=pl.ANY),
            pl.BlockSpec((1, D, H), lambda e: (e, 0, 0)),
            pl.BlockSpec((1, 1, H), lambda e: (e, 0, 0)),
            pl.BlockSpec((1, H, O), lambda e: (e, 0, 0)),
            pl.BlockSpec((1, 1, O), lambda e: (e, 0, 0)),
        ],
        out_specs=pl.BlockSpec((1, CAP, O), lambda e: (e, 0, 0)),
        out_shape=jax.ShapeDtypeStruct((E, CAP, O), jnp.float32),
        scratch_shapes=[
            pltpu.VMEM((CAP, D), jnp.float32),
            pltpu.VMEM((CAP, D), jnp.float32),
            pltpu.VMEM((CAP, D), jnp.float32),
            pltpu.SemaphoreType.DMA,
            pltpu.SemaphoreType.DMA,
            pltpu.SemaphoreType.DMA,
        ],
    )(slot_tok, x, W1, b1.reshape(E, 1, H), W2, b2.reshape(E, 1, O))


def _dispatch_body(idx1_h, idx2_h, loc1_h, loc2p_h, g1_h, g2_h, c1_h,
                   tok_h, pos1_h, pos2_h, s1_h, s2_h,
                   idx1_v, idx2_v, loc1_v, loc2p_v, g1_v, g2_v, c1_v,
                   table, p1b, p2b, s1b, s2b):
    wid = lax.axis_index("s") * NC + lax.axis_index("c")
    slot_lo = wid * SPW
    tok_lo = wid * TPW

    pltpu.sync_copy(idx1_h, idx1_v)
    pltpu.sync_copy(idx2_h, idx2_v)
    pltpu.sync_copy(loc1_h, loc1_v)
    pltpu.sync_copy(loc2p_h, loc2p_v)
    pltpu.sync_copy(g1_h.at[pl.ds(tok_lo, TPW)], g1_v)
    pltpu.sync_copy(g2_h.at[pl.ds(tok_lo, TPW)], g2_v)
    pltpu.sync_copy(c1_h, c1_v)

    def _init(j, _):
        table[pl.ds(j * L, L)] = jnp.zeros((L,), jnp.int32)
        return 0
    lax.fori_loop(0, SPW // L, _init, 0)

    lanes = lax.iota(jnp.int32, L)

    def _scan(j, _):
        sl = pl.ds(j * L, L)
        tvec = lanes + j * L
        i1 = idx1_v[sl]
        l1 = loc1_v[sl]
        p1 = i1 * CAP + l1
        m1 = (l1 < CAP) & (p1 >= slot_lo) & (p1 < slot_lo + SPW)
        plsc.store_scatter(table, [jnp.where(m1, p1 - slot_lo, 0)], tvec,
                           mask=m1)
        i2 = idx2_v[sl]
        l2 = loc2p_v[sl] + plsc.load_gather(c1_v, [i2])
        p2 = i2 * CAP + l2
        m2 = (l2 < CAP) & (p2 >= slot_lo) & (p2 < slot_lo + SPW)
        plsc.store_scatter(table, [jnp.where(m2, p2 - slot_lo, 0)], tvec,
                           mask=m2)
        return 0
    lax.fori_loop(0, S // L, _scan, 0)

    def _scales(j, _):
        slg = pl.ds(tok_lo + j * L, L)
        sll = pl.ds(j * L, L)
        i1 = idx1_v[slg]
        l1 = loc1_v[slg]
        i2 = idx2_v[slg]
        l2 = loc2p_v[slg] + plsc.load_gather(c1_v, [i2])
        g1 = g1_v[sll]
        g2 = g2_v[sll]
        k1 = l1 < CAP
        k2 = l2 < CAP
        g1k = jnp.where(k1, g1, 0.0)
        g2k = jnp.where(k2, g2, 0.0)
        denom = jnp.maximum(g1k + g2k, 1e-9)
        p1b[sll] = i1 * CAP + jnp.where(k1, l1, 0)
        p2b[sll] = i2 * CAP + jnp.where(k2, l2, 0)
        s1b[sll] = g1k / denom
        s2b[sll] = g2k / denom
        return 0
    lax.fori_loop(0, TPW // L, _scales, 0)
    pltpu.sync_copy(p1b, pos1_h.at[pl.ds(tok_lo, TPW)])
    pltpu.sync_copy(p2b, pos2_h.at[pl.ds(tok_lo, TPW)])
    pltpu.sync_copy(s1b, s1_h.at[pl.ds(tok_lo, TPW)])
    pltpu.sync_copy(s2b, s2_h.at[pl.ds(tok_lo, TPW)])

    pltpu.sync_copy(table, tok_h.at[pl.ds(slot_lo, SPW)])


def _dispatch(idx1, idx2, loc1, loc2p, g1, g2, c1):
    f = functools.partial(
        pl.kernel,
        out_type=(
            jax.ShapeDtypeStruct((SLOTS,), jnp.int32),
            jax.ShapeDtypeStruct((S,), jnp.int32),
            jax.ShapeDtypeStruct((S,), jnp.int32),
            jax.ShapeDtypeStruct((S,), jnp.float32),
            jax.ShapeDtypeStruct((S,), jnp.float32),
        ),
        mesh=_MESH,
        scratch_types=[
            pltpu.VMEM((S,), jnp.int32),
            pltpu.VMEM((S,), jnp.int32),
            pltpu.VMEM((S,), jnp.int32),
            pltpu.VMEM((S,), jnp.int32),
            pltpu.VMEM((TPW,), jnp.float32),
            pltpu.VMEM((TPW,), jnp.float32),
            pltpu.VMEM((E,), jnp.int32),
            pltpu.VMEM((SPW,), jnp.int32),
            pltpu.VMEM((TPW,), jnp.int32),
            pltpu.VMEM((TPW,), jnp.int32),
            pltpu.VMEM((TPW,), jnp.float32),
            pltpu.VMEM((TPW,), jnp.float32),
        ],
        compiler_params=pltpu.CompilerParams(needs_layout_passes=False),
    )(_dispatch_body)
    return f(idx1, idx2, loc1, loc2p, g1, g2, c1)


def _combine_body(eo_h, pos1_h, pos2_h, s1_h, s2_h, out_h,
                  p1v, p2v, s1v, s2v, r1, r2, ob, gsem, osem):
    wid = lax.axis_index("s") * NC + lax.axis_index("c")
    tok_lo = wid * TPW

    pltpu.sync_copy(pos1_h.at[pl.ds(tok_lo, TPW)], p1v)
    pltpu.sync_copy(pos2_h.at[pl.ds(tok_lo, TPW)], p2v)
    pltpu.sync_copy(s1_h.at[pl.ds(tok_lo, TPW)], s1v)
    pltpu.sync_copy(s2_h.at[pl.ds(tok_lo, TPW)], s2v)

    nc = TPW // CCH
    g1cp = [None] * 2
    g2cp = [None] * 2
    ocp = [None] * 2
    g1cp[0] = pltpu.async_copy(eo_h.at[p1v[pl.ds(0, CCH)]], r1.at[0], gsem)
    g2cp[0] = pltpu.async_copy(eo_h.at[p2v[pl.ds(0, CCH)]], r2.at[0], gsem)
    for ch in range(nc):
        b = ch % 2
        if ch + 1 < nc:
            nb = 1 - b
            g1cp[nb] = pltpu.async_copy(
                eo_h.at[p1v[pl.ds((ch + 1) * CCH, CCH)]], r1.at[nb], gsem)
            g2cp[nb] = pltpu.async_copy(
                eo_h.at[p2v[pl.ds((ch + 1) * CCH, CCH)]], r2.at[nb], gsem)
        g1cp[b].wait()
        g2cp[b].wait()
        if ocp[b] is not None:
            ocp[b].wait()

        def _tok(t, _):
            ti = jnp.full((L,), ch * CCH + t, jnp.int32)
            sa = plsc.load_gather(s1v, [ti])
            sb = plsc.load_gather(s2v, [ti])

            def _col(v, __):
                sl = pl.ds(v * L, L)
                ob[b, t, sl] = sa * r1[b, t, sl] + sb * r2[b, t, sl]
                return 0
            lax.fori_loop(0, O // L, _col, 0)
            return 0
        lax.fori_loop(0, CCH, _tok, 0)
        ocp[b] = pltpu.async_copy(
            ob.at[b], out_h.at[pl.ds(tok_lo + ch * CCH, CCH)], osem)
    for b in range(2):
        if ocp[b] is not None:
            ocp[b].wait()


def _combine(eo, pos1, pos2, s1, s2):
    f = functools.partial(
        pl.kernel,
        out_type=jax.ShapeDtypeStruct((S, O), jnp.float32),
        mesh=_MESH,
        scratch_types=[
            pltpu.VMEM((TPW,), jnp.int32),
            pltpu.VMEM((TPW,), jnp.int32),
            pltpu.VMEM((TPW,), jnp.float32),
            pltpu.VMEM((TPW,), jnp.float32),
            pltpu.VMEM((2, CCH, O), jnp.float32),
            pltpu.VMEM((2, CCH, O), jnp.float32),
            pltpu.VMEM((2, CCH, O), jnp.float32),
            pltpu.SemaphoreType.DMA,
            pltpu.SemaphoreType.DMA,
        ],
        compiler_params=pltpu.CompilerParams(needs_layout_passes=False),
    )(_combine_body)
    return f(eo, pos1, pos2, s1, s2)


def kernel(x, wg, W1, b1, W2, b2):
    idx1, idx2, loc1, loc2p, g1, g2, c1, laux = _route(x, wg)
    slot_tok, pos1, pos2, s1, s2 = _dispatch(
        idx1.reshape(S), idx2.reshape(S), loc1.reshape(S), loc2p.reshape(S),
        g1.reshape(S), g2.reshape(S), c1.reshape(E))
    eo = _ffn(slot_tok, x, W1, b1, W2, b2)
    out = _combine(eo.reshape(SLOTS, O), pos1, pos2, s1, s2)
    return out, laux[0, 0]
